# baseline (speedup 1.0000x reference)
import functools

import jax
import jax.numpy as jnp
from jax import lax
from jax.experimental import pallas as pl
from jax.experimental.pallas import tpu as pltpu
from jax.experimental.pallas import tpu_sc as plsc

_NC = 2
_NS = 16
_NW = _NC * _NS
_LANES = 16


@functools.partial(jax.jit, static_argnames=("batch", "nrows"))
def _irt_sc(skills, packed, *, batch, nrows):
    b_per_w = batch // _NW
    half = b_per_w // 2
    mesh = plsc.VectorSubcoreMesh(
        core_axis_name="c", subcore_axis_name="s",
        num_cores=_NC, num_subcores=_NS)

    @functools.partial(
        pl.kernel,
        out_type=jax.ShapeDtypeStruct((batch,), jnp.float32),
        mesh=mesh,
        scratch_types=[
            pltpu.VMEM((b_per_w,), jnp.int32),
            pltpu.VMEM((b_per_w,), jnp.int32),
            pltpu.VMEM((b_per_w,), jnp.float32),
            pltpu.VMEM((b_per_w,), jnp.float32),
            pltpu.VMEM((_LANES,), jnp.float32),
            pltpu.SemaphoreType.DMA,
            pltpu.SemaphoreType.DMA,
        ],
    )
    def k(skills_hbm, packed_hbm, out_hbm,
          idx_v, idx2_v, diff_v, disc_v, ab_v, sem0, sem1):
        wid = lax.axis_index("s") * _NC + lax.axis_index("c")
        base = wid * b_per_w
        lo = pl.ds(0, half)
        hi = pl.ds(half, half)
        pltpu.sync_copy(skills_hbm.at[pl.ds(base, b_per_w)], idx_v)
        cp_d0 = pltpu.async_copy(packed_hbm.at[idx_v.at[lo]], diff_v.at[lo], sem0)
        cp_d1 = pltpu.async_copy(packed_hbm.at[idx_v.at[hi]], diff_v.at[hi], sem1)
        for i in range(b_per_w // _LANES):
            sl = pl.ds(i * _LANES, _LANES)
            idx2_v[sl] = idx_v[sl] + nrows
        cp_c0 = pltpu.async_copy(packed_hbm.at[idx2_v.at[lo]], disc_v.at[lo], sem0)
        cp_c1 = pltpu.async_copy(packed_hbm.at[idx2_v.at[hi]], disc_v.at[hi], sem1)
        pltpu.sync_copy(packed_hbm.at[pl.ds(2 * nrows, _LANES)], ab_v)
        a = ab_v[:]
        cp_d0.wait()
        cp_c0.wait()
        for i in range(half // _LANES):
            sl = pl.ds(i * _LANES, _LANES)
            x = disc_v[sl] * (a - diff_v[sl])
            diff_v[sl] = 1.0 / (1.0 + jnp.exp(-x))
        cp_d1.wait()
        cp_c1.wait()
        for i in range(half // _LANES, b_per_w // _LANES):
            sl = pl.ds(i * _LANES, _LANES)
            x = disc_v[sl] * (a - diff_v[sl])
            diff_v[sl] = 1.0 / (1.0 + jnp.exp(-x))
        pltpu.sync_copy(diff_v, out_hbm.at[pl.ds(base, b_per_w)])

    return k(skills, packed)


def kernel(skills, ability_table, difficulty_table, discrimination_table):
    batch = skills.shape[0]
    nrows = difficulty_table.shape[0]
    if skills.dtype != jnp.int32:
        skills = skills.astype(jnp.int32)
    packed = jnp.concatenate([
        difficulty_table.reshape(-1),
        discrimination_table.reshape(-1),
        jnp.broadcast_to(ability_table.reshape(()), (_LANES,)),
    ])
    out = _irt_sc(skills, packed, batch=batch, nrows=nrows)
    return out.reshape(batch, 1)


# speedup vs baseline: 1.1774x; 1.1774x over previous
import functools

import jax
import jax.numpy as jnp
from jax import lax
from jax.experimental import pallas as pl
from jax.experimental.pallas import tpu as pltpu
from jax.experimental.pallas import tpu_sc as plsc

_NC = 2
_NS = 16
_NW = _NC * _NS
_LANES = 16


@functools.partial(jax.jit, static_argnames=("batch",))
def _irt_sc(skills, ability16, difficulty, discrimination, *, batch):
    b_per_w = batch // _NW
    half = b_per_w // 2
    mesh = plsc.VectorSubcoreMesh(
        core_axis_name="c", subcore_axis_name="s",
        num_cores=_NC, num_subcores=_NS)

    @functools.partial(
        pl.kernel,
        out_type=jax.ShapeDtypeStruct((batch,), jnp.float32),
        mesh=mesh,
        scratch_types=[
            pltpu.VMEM((b_per_w,), jnp.int32),
            pltpu.VMEM((b_per_w,), jnp.float32),
            pltpu.VMEM((b_per_w,), jnp.float32),
            pltpu.VMEM((_LANES,), jnp.float32),
            pltpu.SemaphoreType.DMA,
            pltpu.SemaphoreType.DMA,
            pltpu.SemaphoreType.DMA,
        ],
    )
    def k(skills_hbm, ab_hbm, diff_hbm, disc_hbm, out_hbm,
          idx_v, diff_v, disc_v, ab_v, sem0, sem1, semo):
        wid = lax.axis_index("s") * _NC + lax.axis_index("c")
        base = wid * b_per_w
        lo = pl.ds(0, half)
        hi = pl.ds(half, half)
        pltpu.sync_copy(skills_hbm.at[pl.ds(base, half)], idx_v.at[lo])
        cp0 = [
            pltpu.async_copy(diff_hbm.at[idx_v.at[lo]], diff_v.at[lo], sem0),
            pltpu.async_copy(disc_hbm.at[idx_v.at[lo]], disc_v.at[lo], sem0),
        ]
        pltpu.sync_copy(skills_hbm.at[pl.ds(base + half, half)], idx_v.at[hi])
        cp1 = [
            pltpu.async_copy(diff_hbm.at[idx_v.at[hi]], diff_v.at[hi], sem1),
            pltpu.async_copy(disc_hbm.at[idx_v.at[hi]], disc_v.at[hi], sem1),
        ]
        pltpu.sync_copy(ab_hbm, ab_v)
        a = ab_v[:]
        for cp in cp0:
            cp.wait()
        for i in range(half // _LANES):
            sl = pl.ds(i * _LANES, _LANES)
            x = disc_v[sl] * (a - diff_v[sl])
            diff_v[sl] = 1.0 / (1.0 + jnp.exp(-x))
        wr0 = pltpu.async_copy(
            diff_v.at[lo], out_hbm.at[pl.ds(base, half)], semo)
        for cp in cp1:
            cp.wait()
        for i in range(half // _LANES, b_per_w // _LANES):
            sl = pl.ds(i * _LANES, _LANES)
            x = disc_v[sl] * (a - diff_v[sl])
            diff_v[sl] = 1.0 / (1.0 + jnp.exp(-x))
        wr1 = pltpu.async_copy(
            diff_v.at[hi], out_hbm.at[pl.ds(base + half, half)], semo)
        wr0.wait()
        wr1.wait()

    return k(skills, ability16, difficulty, discrimination)


def kernel(skills, ability_table, difficulty_table, discrimination_table):
    batch = skills.shape[0]
    if skills.dtype != jnp.int32:
        skills = skills.astype(jnp.int32)
    ability16 = jnp.broadcast_to(ability_table.reshape(()), (_LANES,))
    diff = difficulty_table.reshape(-1)
    disc = discrimination_table.reshape(-1)
    out = _irt_sc(skills, ability16, diff, disc, batch=batch)
    return out.reshape(batch, 1)
